# baseline (speedup 1.0000x reference)
import jax
import jax.numpy as jnp
from jax import lax
from jax.experimental import pallas as pl
from jax.experimental.pallas import tpu as pltpu
from jax.experimental.pallas import tpu_sc as plsc

M = 100000
B = 16384
D = 128
MOM = 0.8

NC, NS, L = 2, 16, 16
NW = NC * NS
W = 200
TOTWIN = M // W
NBASE = TOTWIN // NW
EXTRA = TOTWIN % NW
IDXC = 2048
NCHUNK = B // IDXC
KPW = (W + L - 1) // L
SSTR = KPW * L
UCAP = SSTR
SLOT_SZ = (NBASE + 1) * SSTR + L
PADROW = W


_GATHER_DNUMS = lax.GatherDimensionNumbers(
    offset_dims=(), collapsed_slice_dims=(0,), start_index_map=(0,))


def _gather16(x, idx):
    return lax.gather(x, idx[:, None], _GATHER_DNUMS, slice_sizes=(1,),
                      mode=lax.GatherScatterMode.PROMISE_IN_BOUNDS)


def _prefix_incl(x, lanes):
    c = x
    for s in (1, 2, 4, 8):
        sh = _gather16(c, jnp.maximum(lanes - s, 0))
        c = c + jnp.where(lanes >= s, sh, jnp.int32(0))
    return c


def _body(mem_hbm, val_hbm, idx_hbm, out_hbm,
          idxbuf, rowl, posl, slot, win, valbuf, wrow, wpos, sem):
    i32 = jnp.int32
    wid = lax.axis_index("s") * NC + lax.axis_index("c")
    nwin = jnp.where(wid < EXTRA, NBASE + 1, NBASE)
    win0 = wid * (NBASE + 1) - jnp.maximum(0, wid - EXTRA)
    lo = win0 * W
    rng = nwin * W
    lanes = lax.iota(i32, L)
    neg1 = jnp.full((L,), -1, i32)

    def init_slot(i, _):
        slot[pl.ds(i * L, L)] = neg1
        return 0
    lax.fori_loop(0, SLOT_SZ // L, init_slot, 0)

    def chunk_body(c, _):
        pltpu.sync_copy(idx_hbm.at[pl.ds(c * IDXC, IDXC)], idxbuf)

        def scan_vreg(i, off):
            v = idxbuf[pl.ds(i * L, L)]
            m = (v >= lo) & (v < lo + rng)
            mi = jnp.where(m, jnp.int32(1), jnp.int32(0))
            csum = _prefix_incl(mi, lanes)
            dest = off + csum - mi
            rl = v - lo
            widx = (rl // W) * SSTR + rl % W
            pos = c * IDXC + i * L + lanes
            plsc.store_scatter(rowl, [dest], widx, mask=m)
            plsc.store_scatter(posl, [dest], pos, mask=m)
            return off + csum[L - 1]
        cnt = lax.fori_loop(0, IDXC // L, scan_vreg, jnp.int32(0))

        def dedup(i, _):
            rv = rowl[pl.ds(i * L, L)]
            pv = posl[pl.ds(i * L, L)]
            for l in range(L):
                mk = (lanes == l) & (i * L + l < cnt)
                plsc.store_scatter(slot, [rv], pv, mask=mk)
            return 0
        lax.fori_loop(0, (cnt + L - 1) // L, dedup, 0)
        return 0
    lax.fori_loop(0, NCHUNK, chunk_body, 0)

    zero16 = jnp.zeros((L,), i32)
    padrow16 = jnp.full((L,), PADROW, i32)

    def window_body(wi, _):
        row0 = lo + wi * W
        pltpu.sync_copy(mem_hbm.at[pl.ds(row0, W)], win.at[pl.ds(0, W)])

        for k in range(KPW):
            wpos[pl.ds(k * L, L)] = zero16
            wrow[pl.ds(k * L, L)] = padrow16

        def collect(k, u):
            rloc = k * L + lanes
            sv = slot[pl.ds(wi * SSTR + k * L, L)]
            m = (rloc < W) & (sv >= 0)
            mi = jnp.where(m, jnp.int32(1), jnp.int32(0))
            csum = _prefix_incl(mi, lanes)
            dest = u + csum - mi
            plsc.store_scatter(wrow, [dest], rloc, mask=m)
            plsc.store_scatter(wpos, [dest], sv, mask=m)
            return u + csum[L - 1]
        u = lax.fori_loop(0, KPW, collect, jnp.int32(0))
        gu = (u + L - 1) // L

        def gather_vals(g, _):
            pltpu.async_copy(val_hbm.at[wpos.at[pl.ds(g * L, L)]],
                             valbuf.at[pl.ds(g * L, L)], sem).wait()
            return 0
        lax.fori_loop(0, gu, gather_vals, 0)

        def blend(g, _):
            rv = wrow[pl.ds(g * L, L)]
            for l in range(L):
                r = rv[l]
                j = g * L + l
                for t in range(D // L):
                    a = win[r, pl.ds(t * L, L)]
                    b = valbuf[j, pl.ds(t * L, L)]
                    win[r, pl.ds(t * L, L)] = (
                        jnp.float32(1.0 - MOM) * a + jnp.float32(MOM) * b)
            return 0
        lax.fori_loop(0, gu, blend, 0)

        pltpu.sync_copy(win.at[pl.ds(0, W)], out_hbm.at[pl.ds(row0, W)])
        return 0
    lax.fori_loop(0, nwin, window_body, 0)


@jax.jit
def kernel(mem, val, idx):
    mesh = plsc.VectorSubcoreMesh(
        core_axis_name="c", subcore_axis_name="s",
        num_cores=NC, num_subcores=NS)
    f = pl.kernel(
        _body,
        out_type=jax.ShapeDtypeStruct((M, D), jnp.float32),
        mesh=mesh,
        compiler_params=pltpu.CompilerParams(needs_layout_passes=False),
        scratch_types=[
            pltpu.VMEM((IDXC,), jnp.int32),
            pltpu.VMEM((IDXC + L,), jnp.int32),
            pltpu.VMEM((IDXC + L,), jnp.int32),
            pltpu.VMEM((SLOT_SZ,), jnp.int32),
            pltpu.VMEM((W + 1, D), jnp.float32),
            pltpu.VMEM((UCAP, D), jnp.float32),
            pltpu.VMEM((UCAP,), jnp.int32),
            pltpu.VMEM((UCAP,), jnp.int32),
            pltpu.SemaphoreType.DMA,
        ],
    )
    return f(mem, val, idx)


# speedup vs baseline: 1.0485x; 1.0485x over previous
import jax
import jax.numpy as jnp
from jax import lax
from jax.experimental import pallas as pl
from jax.experimental.pallas import tpu as pltpu
from jax.experimental.pallas import tpu_sc as plsc

M = 100000
B = 16384
D = 128
MOM = 0.8

NC, NS, L = 2, 16, 16
NW = NC * NS
W = 200
TOTWIN = M // W
NBASE = TOTWIN // NW
EXTRA = TOTWIN % NW
MAXWIN = NBASE + 1
IDXC = 1024
NCHUNK = B // IDXC
KPW = (W + L - 1) // L
SSTR = KPW * L
SLOT_SZ = MAXWIN * SSTR + L
GL_SZ = MAXWIN * SSTR + L
PADROW = W


_GATHER_DNUMS = lax.GatherDimensionNumbers(
    offset_dims=(), collapsed_slice_dims=(0,), start_index_map=(0,))


def _gather16(x, idx):
    return lax.gather(x, idx[:, None], _GATHER_DNUMS, slice_sizes=(1,),
                      mode=lax.GatherScatterMode.PROMISE_IN_BOUNDS)


def _prefix_incl(x, lanes):
    c = x
    for s in (1, 2, 4, 8):
        sh = _gather16(c, jnp.maximum(lanes - s, 0))
        c = c + jnp.where(lanes >= s, sh, jnp.int32(0))
    return c


def _body(mem_hbm, val_hbm, idx_hbm, out_hbm,
          ibuf0, ibuf1, rowl, posl, slot, gwrow, gwpos, ucnt,
          win0, win1, vb0, vb1,
          isem0, isem1, lsem0, lsem1, ssem0, ssem1, gsem0, gsem1):
    i32 = jnp.int32
    f32 = jnp.float32
    wid = lax.axis_index("s") * NC + lax.axis_index("c")
    nwin = jnp.where(wid < EXTRA, NBASE + 1, NBASE)
    win0_idx = wid * (NBASE + 1) - jnp.maximum(0, wid - EXTRA)
    lo = win0_idx * W
    rng = nwin * W
    lanes = lax.iota(i32, L)
    neg1 = jnp.full((L,), -1, i32)

    def init_slot(i, _):
        slot[pl.ds(i * L, L)] = neg1
        return 0
    lax.fori_loop(0, SLOT_SZ // L, init_slot, 0)

    def process_chunk(c, ibuf):
        def scan_vreg(i, off):
            v = ibuf[pl.ds(i * L, L)]
            m = (v >= lo) & (v < lo + rng)
            n = plsc.all_reduce_population_count(m)[0]

            @pl.when(n > 0)
            def _():
                mi = jnp.where(m, jnp.int32(1), jnp.int32(0))
                csum = _prefix_incl(mi, lanes)
                dest = off + csum - mi
                rl = v - lo
                widx = (rl // W) * SSTR + rl % W
                pos = c * IDXC + i * L + lanes
                plsc.store_scatter(rowl, [dest], widx, mask=m)
                plsc.store_scatter(posl, [dest], pos, mask=m)
            return off + n
        cnt = lax.fori_loop(0, IDXC // L, scan_vreg, jnp.int32(0))

        def dedup(i, _):
            rv = rowl[pl.ds(i * L, L)]
            pv = posl[pl.ds(i * L, L)]
            for l in range(L):
                mk = (lanes == l) & (i * L + l < cnt)
                plsc.store_scatter(slot, [rv], pv, mask=mk)
            return 0
        lax.fori_loop(0, (cnt + L - 1) // L, dedup, 0)

    def idx_load(c, ibuf, sem):
        pltpu.async_copy(idx_hbm.at[pl.ds(c * IDXC, IDXC)], ibuf, sem)

    def idx_wait(ibuf, sem):
        pltpu.make_async_copy(idx_hbm.at[pl.ds(0, IDXC)], ibuf, sem).wait()

    idx_load(0, ibuf0, isem0)

    def chunk_pair(t, _):
        c0 = 2 * t
        idx_wait(ibuf0, isem0)
        idx_load(c0 + 1, ibuf1, isem1)
        process_chunk(c0, ibuf0)
        idx_wait(ibuf1, isem1)

        @pl.when(t < NCHUNK // 2 - 1)
        def _():
            idx_load(c0 + 2, ibuf0, isem0)
        process_chunk(c0 + 1, ibuf1)
        return 0
    lax.fori_loop(0, NCHUNK // 2, chunk_pair, 0)

    zero16 = jnp.zeros((L,), i32)
    padrow16 = jnp.full((L,), PADROW, i32)

    def clear_lists(i, _):
        gwrow[pl.ds(i * L, L)] = padrow16
        gwpos[pl.ds(i * L, L)] = zero16
        return 0
    lax.fori_loop(0, GL_SZ // L, clear_lists, 0)

    def collect_win(wi, ucnt_v):
        def collect(k, u):
            rloc = k * L + lanes
            sv = slot[pl.ds(wi * SSTR + k * L, L)]
            m = (rloc < W) & (sv >= 0)
            n = plsc.all_reduce_population_count(m)[0]

            @pl.when(n > 0)
            def _():
                mi = jnp.where(m, jnp.int32(1), jnp.int32(0))
                csum = _prefix_incl(mi, lanes)
                dest = wi * SSTR + u + csum - mi
                plsc.store_scatter(gwrow, [dest], rloc, mask=m)
                plsc.store_scatter(gwpos, [dest], sv, mask=m)
            return u + n
        u = lax.fori_loop(0, KPW, collect, jnp.int32(0))
        return jnp.where(lanes == wi, u, ucnt_v)
    ucnt_v = lax.fori_loop(0, MAXWIN, collect_win, jnp.zeros((L,), i32))
    ucnt[pl.ds(0, L)] = ucnt_v

    def u_of(wi):
        return _gather16(ucnt_v, jnp.full((L,), wi, i32))[0]

    def load_issue(wi, wb, sem):
        pltpu.async_copy(mem_hbm.at[pl.ds(lo + wi * W, W)],
                         wb.at[pl.ds(0, W)], sem)

    def load_wait(wb, sem):
        pltpu.make_async_copy(mem_hbm.at[pl.ds(0, W)],
                              wb.at[pl.ds(0, W)], sem).wait()

    def store_issue(wi, wb, sem):
        pltpu.async_copy(wb.at[pl.ds(0, W)],
                         out_hbm.at[pl.ds(lo + wi * W, W)], sem)

    def store_wait(wb, sem):
        pltpu.make_async_copy(wb.at[pl.ds(0, W)],
                              out_hbm.at[pl.ds(0, W)], sem).wait()

    def gathers_issue(wi, vb, sem):
        gu = (u_of(wi) + L - 1) // L

        def gfn(g, _):
            pltpu.async_copy(val_hbm.at[gwpos.at[pl.ds(wi * SSTR + g * L, L)]],
                             vb.at[pl.ds(g * L, L)], sem)
            return 0
        lax.fori_loop(0, gu, gfn, 0)

    def gathers_drain(wi, vb, sem):
        gu = (u_of(wi) + L - 1) // L

        def dfn(g, _):
            pltpu.make_async_copy(val_hbm.at[pl.ds(0, L)],
                                  vb.at[pl.ds(g * L, L)], sem).wait()
            return 0
        lax.fori_loop(0, gu, dfn, 0)

    def blend(wi, wb, vb):
        gu = (u_of(wi) + L - 1) // L

        def bfn(g, _):
            rv = gwrow[pl.ds(wi * SSTR + g * L, L)]
            for l in range(L):
                r = rv[l]
                j = g * L + l
                for t_ in range(D // L):
                    a = wb[r, pl.ds(t_ * L, L)]
                    b = vb[j, pl.ds(t_ * L, L)]
                    wb[r, pl.ds(t_ * L, L)] = (
                        f32(1.0 - MOM) * a + f32(MOM) * b)
            return 0
        lax.fori_loop(0, gu, bfn, 0)

    load_issue(0, win0, lsem0)
    gathers_issue(0, vb0, gsem0)

    def win_body(wi, wb, vb, lsem, ssem, gsem, qwb, qvb, qlsem, qssem, qgsem):
        @pl.when(wi < nwin)
        def _():
            load_wait(wb, lsem)

            @pl.when(wi + 1 < nwin)
            def _():
                @pl.when(wi >= 1)
                def _():
                    store_wait(qwb, qssem)
                load_issue(wi + 1, qwb, qlsem)
                gathers_issue(wi + 1, qvb, qgsem)
            gathers_drain(wi, vb, gsem)
            blend(wi, wb, vb)
            store_issue(wi, wb, ssem)

    def win_pair(t, _):
        win_body(2 * t, win0, vb0, lsem0, ssem0, gsem0,
                 win1, vb1, lsem1, ssem1, gsem1)
        win_body(2 * t + 1, win1, vb1, lsem1, ssem1, gsem1,
                 win0, vb0, lsem0, ssem0, gsem0)
        return 0
    lax.fori_loop(0, MAXWIN // 2, win_pair, 0)

    store_wait(win0, ssem0)
    store_wait(win1, ssem1)


@jax.jit
def kernel(mem, val, idx):
    mesh = plsc.VectorSubcoreMesh(
        core_axis_name="c", subcore_axis_name="s",
        num_cores=NC, num_subcores=NS)
    f = pl.kernel(
        _body,
        out_type=jax.ShapeDtypeStruct((M, D), jnp.float32),
        mesh=mesh,
        compiler_params=pltpu.CompilerParams(needs_layout_passes=False),
        scratch_types=[
            pltpu.VMEM((IDXC,), jnp.int32),
            pltpu.VMEM((IDXC,), jnp.int32),
            pltpu.VMEM((IDXC + L,), jnp.int32),
            pltpu.VMEM((IDXC + L,), jnp.int32),
            pltpu.VMEM((SLOT_SZ,), jnp.int32),
            pltpu.VMEM((GL_SZ,), jnp.int32),
            pltpu.VMEM((GL_SZ,), jnp.int32),
            pltpu.VMEM((L,), jnp.int32),
            pltpu.VMEM((W + 1, D), jnp.float32),
            pltpu.VMEM((W + 1, D), jnp.float32),
            pltpu.VMEM((SSTR, D), jnp.float32),
            pltpu.VMEM((SSTR, D), jnp.float32),
            pltpu.SemaphoreType.DMA,
            pltpu.SemaphoreType.DMA,
            pltpu.SemaphoreType.DMA,
            pltpu.SemaphoreType.DMA,
            pltpu.SemaphoreType.DMA,
            pltpu.SemaphoreType.DMA,
            pltpu.SemaphoreType.DMA,
            pltpu.SemaphoreType.DMA,
        ],
    )
    return f(mem, val, idx)


# speedup vs baseline: 2.2731x; 2.1680x over previous
import jax
import jax.numpy as jnp
from jax import lax
from jax.experimental import pallas as pl
from jax.experimental.pallas import tpu as pltpu
from jax.experimental.pallas import tpu_sc as plsc

M = 100000
B = 16384
D = 128
MOM = 0.8

NC, NS, L = 2, 16, 16
NW = NC * NS
W = 200
TOTWIN = M // W
NBASE = TOTWIN // NW
EXTRA = TOTWIN % NW
MAXWIN = NBASE + 1
IDXC = 1024
NCHUNK = B // IDXC
KPW = (W + L - 1) // L
SSTR = KPW * L
SLOT_SZ = MAXWIN * SSTR + L
GL_SZ = MAXWIN * SSTR + L
PADROW = W


_GATHER_DNUMS = lax.GatherDimensionNumbers(
    offset_dims=(), collapsed_slice_dims=(0,), start_index_map=(0,))


def _gather16(x, idx):
    return lax.gather(x, idx[:, None], _GATHER_DNUMS, slice_sizes=(1,),
                      mode=lax.GatherScatterMode.PROMISE_IN_BOUNDS)


def _prefix_incl(x, lanes):
    c = x
    for s in (1, 2, 4, 8):
        sh = _gather16(c, jnp.maximum(lanes - s, 0))
        c = c + jnp.where(lanes >= s, sh, jnp.int32(0))
    return c


def _body(mem_hbm, val_hbm, idx_hbm, out_hbm,
          ibuf0, ibuf1, rowl, posl, slot, gwrow, gwpos, ucnt,
          win0, win1, vb0, vb1,
          isem0, isem1, lsem0, lsem1, ssem0, ssem1, gsem0, gsem1):
    i32 = jnp.int32
    f32 = jnp.float32
    wid = lax.axis_index("s") * NC + lax.axis_index("c")
    nwin = jnp.where(wid < EXTRA, NBASE + 1, NBASE)
    win0_idx = wid * (NBASE + 1) - jnp.maximum(0, wid - EXTRA)
    lo = win0_idx * W
    rng = nwin * W
    lanes = lax.iota(i32, L)
    neg1 = jnp.full((L,), -1, i32)

    def init_slot(i, _):
        slot[pl.ds(i * L, L)] = neg1
        return 0
    lax.fori_loop(0, SLOT_SZ // L, init_slot, 0)

    def process_chunk(c, ibuf):
        def scan_vreg(i, off):
            v = ibuf[pl.ds(i * L, L)]
            m = (v >= lo) & (v < lo + rng)
            n = plsc.all_reduce_population_count(m)[0]

            @pl.when(n > 0)
            def _():
                mi = jnp.where(m, jnp.int32(1), jnp.int32(0))
                csum = _prefix_incl(mi, lanes)
                dest = off + csum - mi
                rl = v - lo
                widx = (rl // W) * SSTR + rl % W
                pos = c * IDXC + i * L + lanes
                plsc.store_scatter(rowl, [dest], widx, mask=m)
                plsc.store_scatter(posl, [dest], pos, mask=m)
            return off + n
        cnt = lax.fori_loop(0, IDXC // L, scan_vreg, jnp.int32(0))

        def dedup(i, _):
            rv = rowl[pl.ds(i * L, L)]
            pv = posl[pl.ds(i * L, L)]
            for l in range(L):
                mk = (lanes == l) & (i * L + l < cnt)
                plsc.store_scatter(slot, [rv], pv, mask=mk)
            return 0
        lax.fori_loop(0, (cnt + L - 1) // L, dedup, 0)

    def idx_load(c, ibuf, sem):
        pltpu.async_copy(idx_hbm.at[pl.ds(c * IDXC, IDXC)], ibuf, sem)

    def idx_wait(ibuf, sem):
        pltpu.make_async_copy(idx_hbm.at[pl.ds(0, IDXC)], ibuf, sem).wait()

    idx_load(0, ibuf0, isem0)

    def chunk_pair(t, _):
        c0 = 2 * t
        idx_wait(ibuf0, isem0)
        idx_load(c0 + 1, ibuf1, isem1)
        process_chunk(c0, ibuf0)
        idx_wait(ibuf1, isem1)

        @pl.when(t < NCHUNK // 2 - 1)
        def _():
            idx_load(c0 + 2, ibuf0, isem0)
        process_chunk(c0 + 1, ibuf1)
        return 0
    lax.fori_loop(0, NCHUNK // 2, chunk_pair, 0)

    zero16 = jnp.zeros((L,), i32)
    padrow16 = jnp.full((L,), PADROW, i32)

    def clear_lists(i, _):
        gwrow[pl.ds(i * L, L)] = padrow16
        gwpos[pl.ds(i * L, L)] = i * L + lanes
        return 0
    lax.fori_loop(0, GL_SZ // L, clear_lists, 0)

    def collect_win(wi, ucnt_v):
        def collect(k, u):
            rloc = k * L + lanes
            sv = slot[pl.ds(wi * SSTR + k * L, L)]
            m = (rloc < W) & (sv >= 0)
            n = plsc.all_reduce_population_count(m)[0]

            @pl.when(n > 0)
            def _():
                mi = jnp.where(m, jnp.int32(1), jnp.int32(0))
                csum = _prefix_incl(mi, lanes)
                dest = wi * SSTR + u + csum - mi
                plsc.store_scatter(gwrow, [dest], rloc, mask=m)
                plsc.store_scatter(gwpos, [dest], sv, mask=m)
            return u + n
        u = lax.fori_loop(0, KPW, collect, jnp.int32(0))
        return jnp.where(lanes == wi, u, ucnt_v)
    ucnt_v = lax.fori_loop(0, MAXWIN, collect_win, jnp.zeros((L,), i32))
    ucnt[pl.ds(0, L)] = ucnt_v

    def u_of(wi):
        return _gather16(ucnt_v, jnp.full((L,), wi, i32))[0]

    def load_issue(wi, wb, sem):
        pltpu.async_copy(mem_hbm.at[pl.ds(lo + wi * W, W)],
                         wb.at[pl.ds(0, W)], sem)

    def load_wait(wb, sem):
        pltpu.make_async_copy(mem_hbm.at[pl.ds(0, W)],
                              wb.at[pl.ds(0, W)], sem).wait()

    def store_issue(wi, wb, sem):
        pltpu.async_copy(wb.at[pl.ds(0, W)],
                         out_hbm.at[pl.ds(lo + wi * W, W)], sem)

    def store_wait(wb, sem):
        pltpu.make_async_copy(wb.at[pl.ds(0, W)],
                              out_hbm.at[pl.ds(0, W)], sem).wait()

    def gathers_issue(wi, vb, sem):
        pltpu.async_copy(val_hbm.at[gwpos.at[pl.ds(wi * SSTR, 128)]],
                         vb.at[pl.ds(0, 128)], sem)

        @pl.when(u_of(wi) > 128)
        def _():
            pltpu.async_copy(val_hbm.at[gwpos.at[pl.ds(wi * SSTR + 80, 128)]],
                             vb.at[pl.ds(80, 128)], sem)

    def gathers_drain(wi, vb, sem):
        pltpu.make_async_copy(val_hbm.at[pl.ds(0, 128)],
                              vb.at[pl.ds(0, 128)], sem).wait()

        @pl.when(u_of(wi) > 128)
        def _():
            pltpu.make_async_copy(val_hbm.at[pl.ds(0, 128)],
                                  vb.at[pl.ds(80, 128)], sem).wait()

    def blend(wi, wb, vb):
        gu = (u_of(wi) + L - 1) // L

        def bfn(g, _):
            rv = gwrow[pl.ds(wi * SSTR + g * L, L)]
            for l in range(L):
                r = rv[l]
                j = g * L + l
                for t_ in range(D // L):
                    a = wb[r, pl.ds(t_ * L, L)]
                    b = vb[j, pl.ds(t_ * L, L)]
                    wb[r, pl.ds(t_ * L, L)] = (
                        f32(1.0 - MOM) * a + f32(MOM) * b)
            return 0
        lax.fori_loop(0, gu, bfn, 0)

    load_issue(0, win0, lsem0)
    gathers_issue(0, vb0, gsem0)

    def win_body(wi, wb, vb, lsem, ssem, gsem, qwb, qvb, qlsem, qssem, qgsem):
        @pl.when(wi < nwin)
        def _():
            load_wait(wb, lsem)

            @pl.when(wi + 1 < nwin)
            def _():
                @pl.when(wi >= 1)
                def _():
                    store_wait(qwb, qssem)
                load_issue(wi + 1, qwb, qlsem)
                gathers_issue(wi + 1, qvb, qgsem)
            gathers_drain(wi, vb, gsem)
            blend(wi, wb, vb)
            store_issue(wi, wb, ssem)

    def win_pair(t, _):
        win_body(2 * t, win0, vb0, lsem0, ssem0, gsem0,
                 win1, vb1, lsem1, ssem1, gsem1)
        win_body(2 * t + 1, win1, vb1, lsem1, ssem1, gsem1,
                 win0, vb0, lsem0, ssem0, gsem0)
        return 0
    lax.fori_loop(0, MAXWIN // 2, win_pair, 0)

    store_wait(win0, ssem0)
    store_wait(win1, ssem1)


@jax.jit
def kernel(mem, val, idx):
    mesh = plsc.VectorSubcoreMesh(
        core_axis_name="c", subcore_axis_name="s",
        num_cores=NC, num_subcores=NS)
    f = pl.kernel(
        _body,
        out_type=jax.ShapeDtypeStruct((M, D), jnp.float32),
        mesh=mesh,
        compiler_params=pltpu.CompilerParams(needs_layout_passes=False),
        scratch_types=[
            pltpu.VMEM((IDXC,), jnp.int32),
            pltpu.VMEM((IDXC,), jnp.int32),
            pltpu.VMEM((IDXC + L,), jnp.int32),
            pltpu.VMEM((IDXC + L,), jnp.int32),
            pltpu.VMEM((SLOT_SZ,), jnp.int32),
            pltpu.VMEM((GL_SZ,), jnp.int32),
            pltpu.VMEM((GL_SZ,), jnp.int32),
            pltpu.VMEM((L,), jnp.int32),
            pltpu.VMEM((W + 1, D), jnp.float32),
            pltpu.VMEM((W + 1, D), jnp.float32),
            pltpu.VMEM((SSTR, D), jnp.float32),
            pltpu.VMEM((SSTR, D), jnp.float32),
            pltpu.SemaphoreType.DMA,
            pltpu.SemaphoreType.DMA,
            pltpu.SemaphoreType.DMA,
            pltpu.SemaphoreType.DMA,
            pltpu.SemaphoreType.DMA,
            pltpu.SemaphoreType.DMA,
            pltpu.SemaphoreType.DMA,
            pltpu.SemaphoreType.DMA,
        ],
    )
    return f(mem, val, idx)


# speedup vs baseline: 2.7105x; 1.1924x over previous
import jax
import jax.numpy as jnp
from jax import lax
from jax.experimental import pallas as pl
from jax.experimental.pallas import tpu as pltpu
from jax.experimental.pallas import tpu_sc as plsc

M = 100000
B = 16384
D = 128
MOM = 0.8

NC, NS, L = 2, 16, 16
NW = NC * NS
W = 200
TOTWIN = M // W
NBASE = TOTWIN // NW
EXTRA = TOTWIN % NW
MAXWIN = NBASE + 1
IDXC = 1024
NCHUNK = B // IDXC
KPW = (W + L - 1) // L
SSTR = KPW * L
SLOT_SZ = MAXWIN * SSTR + L
GC = 128
GROWS = (MAXWIN * W + GC - 1) // GC


_GATHER_DNUMS = lax.GatherDimensionNumbers(
    offset_dims=(), collapsed_slice_dims=(0,), start_index_map=(0,))


def _gather16(x, idx):
    return lax.gather(x, idx[:, None], _GATHER_DNUMS, slice_sizes=(1,),
                      mode=lax.GatherScatterMode.PROMISE_IN_BOUNDS)


def _prefix_incl(x, lanes):
    c = x
    for s in (1, 2, 4, 8):
        sh = _gather16(c, jnp.maximum(lanes - s, 0))
        c = c + jnp.where(lanes >= s, sh, jnp.int32(0))
    return c


def _body(mem_hbm, val_hbm, idx_hbm, out_hbm,
          ibuf0, ibuf1, rowl, posl, slot, grow, gpos, wbuf0, wbuf1,
          isem0, isem1, lsem0, lsem1, ssem0, ssem1, gsem0, gsem1):
    i32 = jnp.int32
    f32 = jnp.float32
    wid = lax.axis_index("s") * NC + lax.axis_index("c")
    nwin = jnp.where(wid < EXTRA, NBASE + 1, NBASE)
    first_win = wid * (NBASE + 1) - jnp.maximum(0, wid - EXTRA)
    lo = first_win * W
    rng = nwin * W
    lanes = lax.iota(i32, L)
    neg1 = jnp.full((L,), -1, i32)

    def init_slot(i, _):
        slot[pl.ds(i * L, L)] = neg1
        return 0
    lax.fori_loop(0, SLOT_SZ // L, init_slot, 0)

    def process_chunk(c, ibuf):
        def scan_vreg(i, off):
            v = ibuf[pl.ds(i * L, L)]
            m = (v >= lo) & (v < lo + rng)
            n = plsc.all_reduce_population_count(m)[0]

            @pl.when(n > 0)
            def _():
                mi = jnp.where(m, jnp.int32(1), jnp.int32(0))
                csum = _prefix_incl(mi, lanes)
                dest = off + csum - mi
                rl = v - lo
                widx = (rl // W) * SSTR + rl % W
                pos = c * IDXC + i * L + lanes
                plsc.store_scatter(rowl, [dest], widx, mask=m)
                plsc.store_scatter(posl, [dest], pos, mask=m)
            return off + n
        cnt = lax.fori_loop(0, IDXC // L, scan_vreg, jnp.int32(0))

        def dedup(i, _):
            rv = rowl[pl.ds(i * L, L)]
            pv = posl[pl.ds(i * L, L)]
            for l in range(L):
                mk = (lanes == l) & (i * L + l < cnt)
                plsc.store_scatter(slot, [rv], pv, mask=mk)
            return 0
        lax.fori_loop(0, (cnt + L - 1) // L, dedup, 0)

    def idx_load(c, ibuf, sem):
        pltpu.async_copy(idx_hbm.at[pl.ds(c * IDXC, IDXC)], ibuf, sem)

    def idx_wait(ibuf, sem):
        pltpu.make_async_copy(idx_hbm.at[pl.ds(0, IDXC)], ibuf, sem).wait()

    def load_issue(c, wb, sem):
        pltpu.async_copy(mem_hbm.at[pl.ds(lo + c * W, W)], wb, sem)

    def load_wait(wb, sem):
        pltpu.make_async_copy(mem_hbm.at[pl.ds(0, W)], wb, sem).wait()

    def store_issue(c, wb, sem):
        pltpu.async_copy(wb, out_hbm.at[pl.ds(lo + c * W, W)], sem)

    def store_wait(wb, sem):
        pltpu.make_async_copy(wb, out_hbm.at[pl.ds(0, W)], sem).wait()

    def win_step(c, wb_p, lsem_p, ssem_p, wb_q, lsem_q, ssem_q):
        @pl.when(c < nwin)
        def _():
            load_wait(wb_p, lsem_p)
            store_issue(c, wb_p, ssem_p)

        @pl.when(c + 1 < nwin)
        def _():
            @pl.when(c >= 1)
            def _():
                store_wait(wb_q, ssem_q)
            load_issue(c + 1, wb_q, lsem_q)

    idx_load(0, ibuf0, isem0)
    load_issue(0, wbuf0, lsem0)

    def chunk_pair(t, _):
        c0 = 2 * t
        win_step(c0, wbuf0, lsem0, ssem0, wbuf1, lsem1, ssem1)
        idx_wait(ibuf0, isem0)
        idx_load(c0 + 1, ibuf1, isem1)
        process_chunk(c0, ibuf0)

        win_step(c0 + 1, wbuf1, lsem1, ssem1, wbuf0, lsem0, ssem0)
        idx_wait(ibuf1, isem1)

        @pl.when(t < NCHUNK // 2 - 1)
        def _():
            idx_load(c0 + 2, ibuf0, isem0)
        process_chunk(c0 + 1, ibuf1)
        return 0
    lax.fori_loop(0, NCHUNK // 2, chunk_pair, 0)

    store_wait(wbuf0, ssem0)
    store_wait(wbuf1, ssem1)

    def collect_win(wi, u):
        def collect(k, u):
            rloc = k * L + lanes
            sv = slot[pl.ds(wi * SSTR + k * L, L)]
            m = (rloc < W) & (sv >= 0)
            n = plsc.all_reduce_population_count(m)[0]

            @pl.when(n > 0)
            def _():
                mi = jnp.where(m, jnp.int32(1), jnp.int32(0))
                csum = _prefix_incl(mi, lanes)
                dest = u + csum - mi
                rabs = lo + wi * W + rloc
                plsc.store_scatter(grow, [dest // GC, dest % GC],
                                   rabs, mask=m)
                plsc.store_scatter(gpos, [dest // GC, dest % GC],
                                   sv, mask=m)
            return u + n
        return lax.fori_loop(0, KPW, collect, u)
    u = lax.fori_loop(0, MAXWIN, collect_win, jnp.int32(0))

    @pl.when(u > 0)
    def _():
        r0 = jnp.full((L,), grow[0, pl.ds(0, L)][0], i32)
        p0 = jnp.full((L,), gpos[0, pl.ds(0, L)][0], i32)
        ulim = ((u + GC - 1) // GC) * GC
        for k in range(8):
            dest = u + k * L + lanes
            mk = dest < ulim
            plsc.store_scatter(grow, [dest // GC, dest % GC], r0, mask=mk)
            plsc.store_scatter(gpos, [dest // GC, dest % GC], p0, mask=mk)

        mb = wbuf0.at[pl.ds(0, GC)]
        vb = wbuf1.at[pl.ds(0, GC)]

        def chunk_c(g, _):
            pltpu.async_copy(mem_hbm.at[grow.at[g]], mb, gsem0)
            pltpu.async_copy(val_hbm.at[gpos.at[g]], vb, gsem1)
            pltpu.make_async_copy(mem_hbm.at[pl.ds(0, GC)], mb, gsem0).wait()
            pltpu.make_async_copy(val_hbm.at[pl.ds(0, GC)], vb, gsem1).wait()

            def blend(r, _):
                for t_ in range(D // L):
                    a = wbuf0[r, pl.ds(t_ * L, L)]
                    b = wbuf1[r, pl.ds(t_ * L, L)]
                    wbuf0[r, pl.ds(t_ * L, L)] = (
                        f32(1.0 - MOM) * a + f32(MOM) * b)
                return 0
            lax.fori_loop(0, GC, blend, 0)

            pltpu.async_copy(mb, out_hbm.at[grow.at[g]], gsem0)
            pltpu.make_async_copy(mb, out_hbm.at[pl.ds(0, GC)], gsem0).wait()
            return 0
        lax.fori_loop(0, (u + GC - 1) // GC, chunk_c, 0)


@jax.jit
def kernel(mem, val, idx):
    mesh = plsc.VectorSubcoreMesh(
        core_axis_name="c", subcore_axis_name="s",
        num_cores=NC, num_subcores=NS)
    f = pl.kernel(
        _body,
        out_type=jax.ShapeDtypeStruct((M, D), jnp.float32),
        mesh=mesh,
        compiler_params=pltpu.CompilerParams(needs_layout_passes=False),
        scratch_types=[
            pltpu.VMEM((IDXC,), jnp.int32),
            pltpu.VMEM((IDXC,), jnp.int32),
            pltpu.VMEM((IDXC + L,), jnp.int32),
            pltpu.VMEM((IDXC + L,), jnp.int32),
            pltpu.VMEM((SLOT_SZ,), jnp.int32),
            pltpu.VMEM((GROWS, GC), jnp.int32),
            pltpu.VMEM((GROWS, GC), jnp.int32),
            pltpu.VMEM((W, D), jnp.float32),
            pltpu.VMEM((W, D), jnp.float32),
            pltpu.SemaphoreType.DMA,
            pltpu.SemaphoreType.DMA,
            pltpu.SemaphoreType.DMA,
            pltpu.SemaphoreType.DMA,
            pltpu.SemaphoreType.DMA,
            pltpu.SemaphoreType.DMA,
            pltpu.SemaphoreType.DMA,
            pltpu.SemaphoreType.DMA,
        ],
    )
    return f(mem, val, idx)
